# Initial kernel scaffold; baseline (speedup 1.0000x reference)
import jax
import jax.numpy as jnp
from jax.experimental import pallas as pl


def kernel(features, edge_index, W1, b1, W2, b2, W3, b3):
    raise NotImplementedError("write your pallas kernel here")


# speedup vs baseline: 53.1519x; 53.1519x over previous
import functools

import jax
import jax.numpy as jnp
import numpy as np
from jax import lax
from jax.experimental import pallas as pl
from jax.experimental.pallas import tpu as pltpu
from jax.experimental.pallas import tpu_sc as plsc

N = 10000
RPAD = 10240
NC, NS = 2, 16
CH = 128
ACH = 64
EPT = 10240
EPAD = NS * EPT
EPW = EPAD // (NC * NS)
ANCH = EPT // ACH
DCH = EPW // CH
RPT = RPAD // NS
AZCH = RPT // ACH
ZCH = RPT // CH
TCB = 512
G = RPAD // TCB

_Z = np.int32(0)
_G = np.int32(20)

_MESH = plsc.VectorSubcoreMesh(core_axis_name="c", subcore_axis_name="s")


def _make_agg(width):
  @functools.partial(
      pl.kernel,
      out_type=jax.ShapeDtypeStruct((NC * RPAD, width), jnp.float32),
      mesh=_MESH,
      scratch_types=[
          pltpu.VMEM((EPT,), jnp.int32),
          pltpu.VMEM((ANCH, ACH), jnp.int32),
          pltpu.VMEM((ACH, width), jnp.float32),
          pltpu.VMEM((ACH, width), jnp.float32),
          pltpu.VMEM_SHARED((RPAD, width), jnp.float32),
          pltpu.SemaphoreType.DMA,
          pltpu.SemaphoreType.DMA,
      ],
  )
  def agg_kernel(src_hbm, dst_hbm, zeros_hbm, ht_hbm, out_hbm,
                 src_v, dst_v, buf0, buf1, acc, sem0, sem1):
    i32 = jnp.int32
    cid = lax.axis_index("c")
    sid = lax.axis_index("s")
    wid = cid * i32(NS) + sid
    pltpu.sync_copy(src_hbm.at[wid], src_v)
    pltpu.sync_copy(dst_hbm.at[sid], dst_v)
    row0 = sid * i32(RPT)
    for z in range(AZCH):
      pltpu.sync_copy(zeros_hbm, acc.at[pl.ds(row0 + i32(z * ACH), ACH)])
    plsc.subcore_barrier()

    pltpu.async_copy(ht_hbm.at[src_v.at[pl.ds(0, ACH)]], buf0, sem0)
    pltpu.async_copy(ht_hbm.at[src_v.at[pl.ds(ACH, ACH)]], buf1, sem1)

    @pl.loop(i32(0), i32(ANCH // 2))
    def _main(i):
      for b, (buf, sem) in enumerate(((buf0, sem0), (buf1, sem1))):
        j = i * i32(2) + i32(b)
        pltpu.make_async_copy(ht_hbm.at[pl.ds(0, ACH)], buf, sem).wait()
        pltpu.sync_copy(buf, acc.at[dst_v.at[j]], add=True)

        @pl.when(j + i32(2) < i32(ANCH))
        def _issue():
          pltpu.async_copy(
              ht_hbm.at[src_v.at[pl.ds((j + i32(2)) * i32(ACH), ACH)]],
              buf, sem)

    plsc.subcore_barrier()

    out0 = cid * i32(RPAD) + row0
    for z in range(AZCH):
      pltpu.sync_copy(acc.at[pl.ds(row0 + i32(z * ACH), ACH)], buf0)
      pltpu.sync_copy(buf0, out_hbm.at[pl.ds(out0 + i32(z * ACH), ACH)])

  return agg_kernel


_agg128 = _make_agg(128)


def _norm(d):
  deg = d[:, 0:1]
  return jnp.where(deg > 0, lax.rsqrt(deg), 0.0)


def _tc_first(x, dego, W):
  d_out = W.shape[1]
  half = d_out // 2

  def body(x_ref, d_ref, w_ref, o_ref):
    norm = _norm(d_ref[...])
    h = x_ref[...] * norm
    ht = jnp.dot(h, w_ref[...], preferred_element_type=jnp.float32)
    o_ref[0] = ht[:, :half]
    o_ref[1] = ht[:, half:]

  return pl.pallas_call(
      body,
      grid=(G,),
      in_specs=[
          pl.BlockSpec((TCB, x.shape[1]), lambda i: (i, _Z)),
          pl.BlockSpec((TCB, 128), lambda i: (i, _Z)),
          pl.BlockSpec(W.shape, lambda i: (_Z, _Z)),
      ],
      out_specs=pl.BlockSpec((2, TCB, half), lambda i: (_Z, i, _Z)),
      out_shape=jax.ShapeDtypeStruct((2, RPAD, half), jnp.float32),
  )(x, dego, W)


def _tc_mid(agg, dego, b, W, in_half):
  d_in = agg.shape[1]
  d_out = W.shape[1]
  half = d_out // 2

  def body(xa_ref, xb_ref, d_ref, b_ref, w_ref, o_ref):
    norm = _norm(d_ref[...])
    x = jnp.concatenate([xa_ref[:, :in_half], xb_ref[:, :in_half]], axis=1)
    h = jnp.maximum(x * norm + b_ref[...], 0.0) * norm
    ht = jnp.dot(h, w_ref[...], preferred_element_type=jnp.float32)
    if half < 128:
      zpad = jnp.zeros((TCB, 128 - half), jnp.float32)
      o_ref[0] = jnp.concatenate([ht[:, :half], zpad], axis=1)
      o_ref[1] = jnp.concatenate([ht[:, half:], zpad], axis=1)
    else:
      o_ref[0] = ht[:, :half]
      o_ref[1] = ht[:, half:]

  return pl.pallas_call(
      body,
      grid=(G,),
      in_specs=[
          pl.BlockSpec((TCB, d_in), lambda i: (i, _Z)),
          pl.BlockSpec((TCB, d_in), lambda i: (_G + i, _Z)),
          pl.BlockSpec((TCB, 128), lambda i: (i, _Z)),
          pl.BlockSpec(b.shape, lambda i: (_Z, _Z)),
          pl.BlockSpec(W.shape, lambda i: (_Z, _Z)),
      ],
      out_specs=pl.BlockSpec((2, TCB, 128), lambda i: (_Z, i, _Z)),
      out_shape=jax.ShapeDtypeStruct((2, RPAD, 128), jnp.float32),
  )(agg, agg, dego, b, W)


def _tc_last(agg, dego, b, in_half):
  d_in = agg.shape[1]

  def body(xa_ref, xb_ref, d_ref, b_ref, o_ref):
    norm = _norm(d_ref[...])
    x = jnp.concatenate([xa_ref[:, :in_half], xb_ref[:, :in_half]], axis=1)
    o_ref[...] = x * norm + b_ref[...]

  return pl.pallas_call(
      body,
      grid=(G,),
      in_specs=[
          pl.BlockSpec((TCB, d_in), lambda i: (i, _Z)),
          pl.BlockSpec((TCB, d_in), lambda i: (_G + i, _Z)),
          pl.BlockSpec((TCB, 128), lambda i: (i, _Z)),
          pl.BlockSpec(b.shape, lambda i: (_Z, _Z)),
      ],
      out_specs=pl.BlockSpec((TCB, 2 * in_half), lambda i: (i, _Z)),
      out_shape=jax.ShapeDtypeStruct((RPAD, 2 * in_half), jnp.float32),
  )(agg, agg, dego, b)


def kernel(features, edge_index, W1, b1, W2, b2, W3, b3):
  W1 = W1.astype(jnp.float32)
  W2 = W2.astype(jnp.float32)
  W3 = W3.astype(jnp.float32)
  b1 = b1.astype(jnp.float32)
  b2 = b2.astype(jnp.float32)
  b3 = b3.astype(jnp.float32)
  features = features.astype(jnp.float32)
  src = edge_index[0].astype(jnp.int32)
  dst = edge_index[1].astype(jnp.int32)
  e = src.shape[0]
  pad = EPAD - e
  srcp = jnp.concatenate([src, jnp.zeros((pad,), jnp.int32)])
  dstp = jnp.concatenate([dst, jnp.full((pad,), N, jnp.int32)])
  src2 = jnp.stack([srcp, srcp + RPAD]).reshape(NC * NS, EPT)
  dst3 = dstp.reshape(NS, ANCH, ACH)

  z128 = jnp.zeros((ACH, 128), jnp.float32)
  ones_ht = jnp.ones((NC * RPAD, 128), jnp.float32)

  dego = _agg128(src2, dst3, z128, ones_ht)
  ht1 = _tc_first(features, dego, W1)
  agg1 = _agg128(src2, dst3, z128, ht1.reshape(NC * RPAD, 128))
  ht2 = _tc_mid(agg1, dego, b1.reshape(1, -1), W2, 128)
  agg2 = _agg128(src2, dst3, z128, ht2.reshape(NC * RPAD, 128))
  ht3 = _tc_mid(agg2, dego, b2.reshape(1, -1), W3, 128)
  agg3 = _agg128(src2, dst3, z128, ht3.reshape(NC * RPAD, 128))
  out = _tc_last(agg3, dego, b3.reshape(1, -1), 64)
  return out[:N].astype(jnp.float64)
